# baseline (speedup 1.0000x reference)
import functools

import jax
import jax.numpy as jnp
from jax import lax
from jax.experimental import pallas as pl
from jax.experimental.pallas import tpu as pltpu
from jax.experimental.pallas import tpu_sc as plsc

EPS = 1e-5
ROW_BLK = 2000
L = 16
NW = 32
NW_ACT = 25
CHUNK = 128


def _sc_dedup_build(m, u):
    nvec = u // L
    vpw = nvec // NW
    extra = nvec - vpw * NW
    vchunk = 122
    nchunks = vpw // vchunk
    kpc = vchunk * L
    mgroups = m // L
    gpt = mgroups // 16
    rem_g = mgroups - gpt * 16
    wpt = gpt * L

    mesh = plsc.VectorSubcoreMesh(core_axis_name="c", subcore_axis_name="s")

    def process_vec(table, kbuf, voff, base_u):
        lanes = lax.broadcasted_iota(jnp.int32, (L,), 0)
        kv = kbuf[pl.ds(voff * L, L)]
        comp = (kv << 4) | lanes
        scomp, slane = plsc.sort_key_val(comp, lanes)
        skey = scomp >> 4
        uval = slane + base_u
        nxt = lax.gather(
            skey, jnp.minimum(lanes + 1, 15)[:, None],
            dimension_numbers=lax.GatherDimensionNumbers(
                offset_dims=(), collapsed_slice_dims=(0,),
                start_index_map=(0,)),
            slice_sizes=(1,), mode=lax.GatherScatterMode.PROMISE_IN_BOUNDS)
        end = (skey != nxt) | (lanes == 15)
        plsc.store_scatter(table, [skey], uval, mask=end)

    def body(ki, tabs_out, table, kbuf, sem):
        cid = lax.axis_index("c")
        sid = lax.axis_index("s")
        wid = sid * 2 + cid

        neg1 = jnp.full((L,), -1, jnp.int32)

        def memset(i, carry):
            for k in range(25):
                table[pl.ds((i * 25 + k) * L, L)] = neg1
            return carry

        lax.fori_loop(0, m // (L * 25), memset, 0)

        start_vec = wid * vpw + jnp.minimum(wid, extra)
        start_key = pl.multiple_of(start_vec * L, 8)
        for ch in range(nchunks):
            off = pl.multiple_of(start_key + ch * kpc, 8)
            pltpu.sync_copy(ki.at[pl.ds(off, kpc)], kbuf)

            def pv(j, carry):
                process_vec(table, kbuf, j, off + j * L)
                return carry

            lax.fori_loop(0, vchunk, pv, 0)

        @pl.when(wid < extra)
        def _():
            off = pl.multiple_of(start_key + vpw * L, 8)
            pltpu.sync_copy(ki.at[pl.ds(off, L)], kbuf.at[pl.ds(0, L)])
            process_vec(table, kbuf, 0, off)

        pltpu.sync_copy(table, tabs_out.at[pl.ds(wid * m, m)])

    return pl.kernel(
        body,
        out_type=jax.ShapeDtypeStruct((NW * m,), jnp.int32),
        mesh=mesh,
        compiler_params=pltpu.CompilerParams(needs_layout_passes=False),
        scratch_types=[
            pltpu.VMEM((m,), jnp.int32),
            pltpu.VMEM((kpc,), jnp.int32),
            pltpu.SemaphoreType.DMA,
        ],
    )


def _sc_gather_build(m, d, u, ncfg):
    rpw = 3120
    rem0 = m - NW * rpw
    mesh = plsc.VectorSubcoreMesh(core_axis_name="c", subcore_axis_name="s")

    NBUF = 4
    LA = 2

    def make_chunks(nrows):
        chunks = []
        off = 0
        while off < nrows:
            n = min(CHUNK, nrows - off)
            chunks.append((off, n))
            off += n
        return chunks

    def run_range(tabs, vi, cfg, g_out, win_out, r0, nrows,
                  acc, tb, wc_flat, vsel, rows, msem, gsem, wsem, vsem):
        pltpu.sync_copy(tabs.at[pl.ds(r0, nrows)], acc.at[pl.ds(0, nrows)])
        nv = nrows // L
        unr = 13 if nv % 13 == 0 else (10 if nv % 10 == 0 else 1)

        dprev = pltpu.async_copy(tabs.at[pl.ds(1 * m + r0, nrows)],
                                 tb[1 % 2].at[pl.ds(0, nrows)], msem)
        for t in range(1, NW):
            if t + 1 < NW:
                dnext = pltpu.async_copy(
                    tabs.at[pl.ds((t + 1) * m + r0, nrows)],
                    tb[(t + 1) % 2].at[pl.ds(0, nrows)], msem)
            dprev.wait()
            cur = tb[t % 2]

            def mx(j, c2, cur=cur):
                for k in range(unr):
                    o = (j * unr + k) * L
                    acc[pl.ds(o, L)] = jnp.maximum(acc[pl.ds(o, L)],
                                                   cur[pl.ds(o, L)])
                return c2

            lax.fori_loop(0, nv // unr, mx, 0)
            if t + 1 < NW:
                dprev = dnext

        def clamp_body(i, carry):
            for k in range(unr):
                o = (i * unr + k) * L
                wc_flat[pl.ds(o, L)] = jnp.maximum(acc[pl.ds(o, L)], 0)
            return carry

        lax.fori_loop(0, nv // unr, clamp_body, 0)
        wout = pltpu.async_copy(acc.at[pl.ds(0, nrows)],
                                win_out.at[pl.ds(r0, nrows)], msem)

        chunks = make_chunks(nrows)
        vds = [pltpu.async_copy(vi.at[wc_flat.at[pl.ds(off, n)]],
                                vsel.at[pl.ds(off, n)], vsem)
               for off, n in chunks]
        for vd in vds:
            vd.wait()

        nch = len(chunks)
        gds = [None] * nch
        wds = [None] * nch
        dsts = [None] * nch
        for i in range(nch + LA):
            if i < nch:
                off, n = chunks[i]
                b = i % NBUF
                if i >= NBUF:
                    wds[i - NBUF].wait()
                dsts[i] = rows.at[b, pl.ds(0, n), :]
                gds[i] = pltpu.async_copy(
                    cfg.at[vsel.at[pl.ds(off, n)]], dsts[i], gsem)
            j = i - LA
            if 0 <= j < nch:
                off, n = chunks[j]
                gds[j].wait()
                wds[j] = pltpu.async_copy(
                    dsts[j], g_out.at[pl.ds(r0 + off, n), :], wsem)
        for j in range(max(nch - NBUF, 0), nch):
            wds[j].wait()
        wout.wait()

    def body(tabs, vi, cfg, g_out, win_out, acc, tb0, tb1, wc_flat, vsel,
             rows, msem, gsem, wsem, vsem):
        cid = lax.axis_index("c")
        sid = lax.axis_index("s")
        wid = sid * 2 + cid
        r0 = wid * rpw
        run_range(tabs, vi, cfg, g_out, win_out, r0, rpw,
                  acc, [tb0, tb1], wc_flat, vsel, rows,
                  msem, gsem, wsem, vsem)

        @pl.when(wid == 0)
        def _():
            run_range(tabs, vi, cfg, g_out, win_out, NW * rpw, rem0,
                      acc, [tb0, tb1], wc_flat, vsel, rows,
                      msem, gsem, wsem, vsem)

    return pl.kernel(
        body,
        out_type=[
            jax.ShapeDtypeStruct((m, d), jnp.float32),
            jax.ShapeDtypeStruct((m,), jnp.int32),
        ],
        mesh=mesh,
        scratch_types=[
            pltpu.VMEM((3120,), jnp.int32),
            pltpu.VMEM((3120,), jnp.int32),
            pltpu.VMEM((3120,), jnp.int32),
            pltpu.VMEM((3120,), jnp.int32),
            pltpu.VMEM((3120,), jnp.int32),
            pltpu.VMEM((NBUF, CHUNK, d), jnp.float32),
            pltpu.SemaphoreType.DMA,
            pltpu.SemaphoreType.DMA,
            pltpu.SemaphoreType.DMA,
            pltpu.SemaphoreType.DMA,
        ],
    )


def _dense_body(prev_ref, g_ref, win_ref, wzp_ref, wzu_ref, wc_ref,
                bz_ref, bc_ref, gamma_ref, beta_ref, out_ref):
    prev = prev_ref[...]
    g = g_ref[...]
    a = jnp.dot(prev, wzp_ref[...], preferred_element_type=jnp.float32)
    b = jnp.dot(g, wzu_ref[...], preferred_element_type=jnp.float32)
    c = jnp.dot(g, wc_ref[...], preferred_element_type=jnp.float32)
    z = jax.nn.sigmoid(a + b + bz_ref[...])
    cand = jax.nn.relu(c + bc_ref[...])
    new = z * prev + (1.0 - z) * cand
    mask = win_ref[...] >= 0
    row = jnp.where(mask, new, prev)
    mean = jnp.mean(row, axis=-1, keepdims=True)
    var = jnp.mean(jnp.square(row - mean), axis=-1, keepdims=True)
    out_ref[...] = (row - mean) * lax.rsqrt(var + EPS) * gamma_ref[...] + beta_ref[...]


def _dense_update(prev, g, winner, Wz_p, Wz_u, bz, Wc, bc, gamma, beta):
    m, d = prev.shape
    grid = (m // ROW_BLK,)
    row_spec = pl.BlockSpec((ROW_BLK, d), lambda i: (i, 0))
    win_spec = pl.BlockSpec((ROW_BLK, 1), lambda i: (i, 0))
    full_w = pl.BlockSpec((d, d), lambda i: (0, 0))
    vec_spec = pl.BlockSpec((1, d), lambda i: (0, 0))
    return pl.pallas_call(
        _dense_body,
        grid=grid,
        in_specs=[row_spec, row_spec, win_spec, full_w, full_w, full_w,
                  vec_spec, vec_spec, vec_spec, vec_spec],
        out_specs=row_spec,
        out_shape=jax.ShapeDtypeStruct((m, d), jnp.float32),
    )(prev, g, winner, Wz_p, Wz_u, Wc,
      bz[None, :], bc[None, :], gamma[None, :], beta[None, :])


def kernel(previous_ast_nodes_encodings, new_cfg_nodes_encodings,
           Wz_p, Wz_u, bz, Wc, bc, gamma, beta,
           key_indices, value_indices):
    prev = previous_ast_nodes_encodings
    cfg = new_cfg_nodes_encodings
    m, d = prev.shape
    ncfg = cfg.shape[0]
    u = key_indices.shape[0]
    ki = key_indices.astype(jnp.int32)
    vi = value_indices.astype(jnp.int32)

    tabs = _sc_dedup_build(m, u)(ki)
    g, win_merged = _sc_gather_build(m, d, u, ncfg)(tabs, vi, cfg)
    winner_col = win_merged.reshape(m, 1)
    return _dense_update(prev, g, winner_col, Wz_p, Wz_u, bz, Wc, bc,
                         gamma, beta)


# speedup vs baseline: 1.0602x; 1.0602x over previous
import functools

import jax
import jax.numpy as jnp
from jax import lax
from jax.experimental import pallas as pl
from jax.experimental.pallas import tpu as pltpu
from jax.experimental.pallas import tpu_sc as plsc

EPS = 1e-5
ROW_BLK = 10000
L = 16
NW = 32
NW_ACT = 25
CHUNK = 128


def _sc_dedup_build(m, u):
    nvec = u // L
    vpw = nvec // NW
    extra = nvec - vpw * NW
    vchunk = 122
    nchunks = vpw // vchunk
    kpc = vchunk * L
    mgroups = m // L
    gpt = mgroups // 16
    rem_g = mgroups - gpt * 16
    wpt = gpt * L

    mesh = plsc.VectorSubcoreMesh(core_axis_name="c", subcore_axis_name="s")

    def process_vec(table, kbuf, voff, base_u):
        lanes = lax.broadcasted_iota(jnp.int32, (L,), 0)
        kv = kbuf[pl.ds(voff * L, L)]
        comp = (kv << 4) | lanes
        scomp, slane = plsc.sort_key_val(comp, lanes)
        skey = scomp >> 4
        uval = slane + base_u
        nxt = lax.gather(
            skey, jnp.minimum(lanes + 1, 15)[:, None],
            dimension_numbers=lax.GatherDimensionNumbers(
                offset_dims=(), collapsed_slice_dims=(0,),
                start_index_map=(0,)),
            slice_sizes=(1,), mode=lax.GatherScatterMode.PROMISE_IN_BOUNDS)
        end = (skey != nxt) | (lanes == 15)
        plsc.store_scatter(table, [skey], uval, mask=end)

    def body(ki, tabs_out, table, kbuf, sem):
        cid = lax.axis_index("c")
        sid = lax.axis_index("s")
        wid = sid * 2 + cid

        neg1 = jnp.full((L,), -1, jnp.int32)

        def memset(i, carry):
            for k in range(25):
                table[pl.ds((i * 25 + k) * L, L)] = neg1
            return carry

        lax.fori_loop(0, m // (L * 25), memset, 0)

        start_vec = wid * vpw + jnp.minimum(wid, extra)
        start_key = pl.multiple_of(start_vec * L, 8)
        for ch in range(nchunks):
            off = pl.multiple_of(start_key + ch * kpc, 8)
            pltpu.sync_copy(ki.at[pl.ds(off, kpc)], kbuf)

            def pv(j, carry):
                process_vec(table, kbuf, j, off + j * L)
                return carry

            lax.fori_loop(0, vchunk, pv, 0)

        @pl.when(wid < extra)
        def _():
            off = pl.multiple_of(start_key + vpw * L, 8)
            pltpu.sync_copy(ki.at[pl.ds(off, L)], kbuf.at[pl.ds(0, L)])
            process_vec(table, kbuf, 0, off)

        pltpu.sync_copy(table, tabs_out.at[pl.ds(wid * m, m)])

    return pl.kernel(
        body,
        out_type=jax.ShapeDtypeStruct((NW * m,), jnp.int32),
        mesh=mesh,
        compiler_params=pltpu.CompilerParams(needs_layout_passes=False),
        scratch_types=[
            pltpu.VMEM((m,), jnp.int32),
            pltpu.VMEM((kpc,), jnp.int32),
            pltpu.SemaphoreType.DMA,
        ],
    )


def _sc_gather_build(m, d, u, ncfg):
    rpw = 3120
    rem0 = m - NW * rpw
    mesh = plsc.VectorSubcoreMesh(core_axis_name="c", subcore_axis_name="s")

    NBUF = 4
    LA = 2

    def make_chunks(nrows):
        chunks = []
        off = 0
        while off < nrows:
            n = min(CHUNK, nrows - off)
            chunks.append((off, n))
            off += n
        return chunks

    def run_range(tabs, vi, cfg, g_out, win_out, r0, nrows,
                  acc, tb, wc_flat, vsel, rows, msem, gsem, wsem, vsem):
        pltpu.sync_copy(tabs.at[pl.ds(r0, nrows)], acc.at[pl.ds(0, nrows)])
        nv = nrows // L
        unr = 13 if nv % 13 == 0 else (10 if nv % 10 == 0 else 1)

        dprev = pltpu.async_copy(tabs.at[pl.ds(1 * m + r0, nrows)],
                                 tb[1 % 2].at[pl.ds(0, nrows)], msem)
        for t in range(1, NW):
            if t + 1 < NW:
                dnext = pltpu.async_copy(
                    tabs.at[pl.ds((t + 1) * m + r0, nrows)],
                    tb[(t + 1) % 2].at[pl.ds(0, nrows)], msem)
            dprev.wait()
            cur = tb[t % 2]

            def mx(j, c2, cur=cur):
                for k in range(unr):
                    o = (j * unr + k) * L
                    acc[pl.ds(o, L)] = jnp.maximum(acc[pl.ds(o, L)],
                                                   cur[pl.ds(o, L)])
                return c2

            lax.fori_loop(0, nv // unr, mx, 0)
            if t + 1 < NW:
                dprev = dnext

        def clamp_body(i, carry):
            for k in range(unr):
                o = (i * unr + k) * L
                wc_flat[pl.ds(o, L)] = jnp.maximum(acc[pl.ds(o, L)], 0)
            return carry

        lax.fori_loop(0, nv // unr, clamp_body, 0)
        wout = pltpu.async_copy(acc.at[pl.ds(0, nrows)],
                                win_out.at[pl.ds(r0, nrows)], msem)

        chunks = make_chunks(nrows)
        vds = [pltpu.async_copy(vi.at[wc_flat.at[pl.ds(off, n)]],
                                vsel.at[pl.ds(off, n)], vsem)
               for off, n in chunks]
        for vd in vds:
            vd.wait()

        nch = len(chunks)
        gds = [None] * nch
        wds = [None] * nch
        dsts = [None] * nch
        for i in range(nch + LA):
            if i < nch:
                off, n = chunks[i]
                b = i % NBUF
                if i >= NBUF:
                    wds[i - NBUF].wait()
                dsts[i] = rows.at[b, pl.ds(0, n), :]
                gds[i] = pltpu.async_copy(
                    cfg.at[vsel.at[pl.ds(off, n)]], dsts[i], gsem)
            j = i - LA
            if 0 <= j < nch:
                off, n = chunks[j]
                gds[j].wait()
                wds[j] = pltpu.async_copy(
                    dsts[j], g_out.at[pl.ds(r0 + off, n), :], wsem)
        for j in range(max(nch - NBUF, 0), nch):
            wds[j].wait()
        wout.wait()

    def body(tabs, vi, cfg, g_out, win_out, acc, tb0, tb1, wc_flat, vsel,
             rows, msem, gsem, wsem, vsem):
        cid = lax.axis_index("c")
        sid = lax.axis_index("s")
        wid = sid * 2 + cid
        r0 = wid * rpw
        run_range(tabs, vi, cfg, g_out, win_out, r0, rpw,
                  acc, [tb0, tb1], wc_flat, vsel, rows,
                  msem, gsem, wsem, vsem)

        @pl.when(wid == 0)
        def _():
            run_range(tabs, vi, cfg, g_out, win_out, NW * rpw, rem0,
                      acc, [tb0, tb1], wc_flat, vsel, rows,
                      msem, gsem, wsem, vsem)

    return pl.kernel(
        body,
        out_type=[
            jax.ShapeDtypeStruct((m, d), jnp.float32),
            jax.ShapeDtypeStruct((m,), jnp.int32),
        ],
        mesh=mesh,
        scratch_types=[
            pltpu.VMEM((3120,), jnp.int32),
            pltpu.VMEM((3120,), jnp.int32),
            pltpu.VMEM((3120,), jnp.int32),
            pltpu.VMEM((3120,), jnp.int32),
            pltpu.VMEM((3120,), jnp.int32),
            pltpu.VMEM((NBUF, CHUNK, d), jnp.float32),
            pltpu.SemaphoreType.DMA,
            pltpu.SemaphoreType.DMA,
            pltpu.SemaphoreType.DMA,
            pltpu.SemaphoreType.DMA,
        ],
    )


def _dense_body(prev_ref, g_ref, win_ref, wzp_ref, wzu_ref, wc_ref,
                bz_ref, bc_ref, gamma_ref, beta_ref, out_ref):
    prev = prev_ref[...]
    g = g_ref[...]
    a = jnp.dot(prev, wzp_ref[...], preferred_element_type=jnp.float32)
    b = jnp.dot(g, wzu_ref[...], preferred_element_type=jnp.float32)
    c = jnp.dot(g, wc_ref[...], preferred_element_type=jnp.float32)
    z = jax.nn.sigmoid(a + b + bz_ref[...])
    cand = jax.nn.relu(c + bc_ref[...])
    new = z * prev + (1.0 - z) * cand
    mask = win_ref[...] >= 0
    row = jnp.where(mask, new, prev)
    mean = jnp.mean(row, axis=-1, keepdims=True)
    var = jnp.mean(jnp.square(row - mean), axis=-1, keepdims=True)
    out_ref[...] = (row - mean) * lax.rsqrt(var + EPS) * gamma_ref[...] + beta_ref[...]


def _dense_update(prev, g, winner, Wz_p, Wz_u, bz, Wc, bc, gamma, beta):
    m, d = prev.shape
    grid = (m // ROW_BLK,)
    row_spec = pl.BlockSpec((ROW_BLK, d), lambda i: (i, 0))
    win_spec = pl.BlockSpec((ROW_BLK, 1), lambda i: (i, 0))
    full_w = pl.BlockSpec((d, d), lambda i: (0, 0))
    vec_spec = pl.BlockSpec((1, d), lambda i: (0, 0))
    return pl.pallas_call(
        _dense_body,
        grid=grid,
        in_specs=[row_spec, row_spec, win_spec, full_w, full_w, full_w,
                  vec_spec, vec_spec, vec_spec, vec_spec],
        out_specs=row_spec,
        out_shape=jax.ShapeDtypeStruct((m, d), jnp.float32),
    )(prev, g, winner, Wz_p, Wz_u, Wc,
      bz[None, :], bc[None, :], gamma[None, :], beta[None, :])


def kernel(previous_ast_nodes_encodings, new_cfg_nodes_encodings,
           Wz_p, Wz_u, bz, Wc, bc, gamma, beta,
           key_indices, value_indices):
    prev = previous_ast_nodes_encodings
    cfg = new_cfg_nodes_encodings
    m, d = prev.shape
    ncfg = cfg.shape[0]
    u = key_indices.shape[0]
    ki = key_indices.astype(jnp.int32)
    vi = value_indices.astype(jnp.int32)

    tabs = _sc_dedup_build(m, u)(ki)
    g, win_merged = _sc_gather_build(m, d, u, ncfg)(tabs, vi, cfg)
    winner_col = win_merged.reshape(m, 1)
    return _dense_update(prev, g, winner_col, Wz_p, Wz_u, bz, Wc, bc,
                         gamma, beta)


# speedup vs baseline: 1.2479x; 1.1771x over previous
import functools

import jax
import jax.numpy as jnp
from jax import lax
from jax.experimental import pallas as pl
from jax.experimental.pallas import tpu as pltpu
from jax.experimental.pallas import tpu_sc as plsc

EPS = 1e-5
ROW_BLK = 5120
L = 16
NW = 32
NW_ACT = 25
CHUNK = 128


def _sc_dedup_build(m, u):
    nvec = u // L
    vpw = nvec // NW
    extra = nvec - vpw * NW
    vchunk = 122
    nchunks = vpw // vchunk
    kpc = vchunk * L
    mgroups = m // L
    gpt = mgroups // 16
    rem_g = mgroups - gpt * 16
    wpt = gpt * L

    mesh = plsc.VectorSubcoreMesh(core_axis_name="c", subcore_axis_name="s")

    def process_vec(table, kbuf, voff, base_u):
        lanes = lax.broadcasted_iota(jnp.int32, (L,), 0)
        kv = kbuf[pl.ds(voff * L, L)]
        comp = (kv << 4) | lanes
        scomp, slane = plsc.sort_key_val(comp, lanes)
        skey = scomp >> 4
        uval = slane + base_u
        nxt = lax.gather(
            skey, jnp.minimum(lanes + 1, 15)[:, None],
            dimension_numbers=lax.GatherDimensionNumbers(
                offset_dims=(), collapsed_slice_dims=(0,),
                start_index_map=(0,)),
            slice_sizes=(1,), mode=lax.GatherScatterMode.PROMISE_IN_BOUNDS)
        end = (skey != nxt) | (lanes == 15)
        plsc.store_scatter(table, [skey], uval, mask=end)

    def body(ki, tabs_out, table, kbuf, sem):
        cid = lax.axis_index("c")
        sid = lax.axis_index("s")
        wid = sid * 2 + cid

        neg1 = jnp.full((L,), -1, jnp.int32)

        def memset(i, carry):
            for k in range(25):
                table[pl.ds((i * 25 + k) * L, L)] = neg1
            return carry

        lax.fori_loop(0, m // (L * 25), memset, 0)

        start_vec = wid * vpw + jnp.minimum(wid, extra)
        start_key = pl.multiple_of(start_vec * L, 8)
        for ch in range(nchunks):
            off = pl.multiple_of(start_key + ch * kpc, 8)
            pltpu.sync_copy(ki.at[pl.ds(off, kpc)], kbuf)

            def pv(j, carry):
                process_vec(table, kbuf, j, off + j * L)
                return carry

            lax.fori_loop(0, vchunk, pv, 0)

        @pl.when(wid < extra)
        def _():
            off = pl.multiple_of(start_key + vpw * L, 8)
            pltpu.sync_copy(ki.at[pl.ds(off, L)], kbuf.at[pl.ds(0, L)])
            process_vec(table, kbuf, 0, off)

        pltpu.sync_copy(table, tabs_out.at[pl.ds(wid * m, m)])

    return pl.kernel(
        body,
        out_type=jax.ShapeDtypeStruct((NW * m,), jnp.int32),
        mesh=mesh,
        compiler_params=pltpu.CompilerParams(needs_layout_passes=False),
        scratch_types=[
            pltpu.VMEM((m,), jnp.int32),
            pltpu.VMEM((kpc,), jnp.int32),
            pltpu.SemaphoreType.DMA,
        ],
    )


def _sc_gather_build(m, d, u, ncfg):
    rpw = 3120
    rem0 = m - NW * rpw
    mesh = plsc.VectorSubcoreMesh(core_axis_name="c", subcore_axis_name="s")

    NBUF = 4
    LA = 2

    def make_chunks(nrows):
        chunks = []
        off = 0
        while off < nrows:
            n = min(CHUNK, nrows - off)
            chunks.append((off, n))
            off += n
        return chunks

    def run_range(tabs, vi, cfg, g_out, win_out, r0, nrows,
                  acc, tb, wc_flat, vsel, rows, msem, gsem, wsem, vsem):
        pltpu.sync_copy(tabs.at[pl.ds(r0, nrows)], acc.at[pl.ds(0, nrows)])
        nv = nrows // L
        unr = 13 if nv % 13 == 0 else (10 if nv % 10 == 0 else 1)

        dprev = pltpu.async_copy(tabs.at[pl.ds(1 * m + r0, nrows)],
                                 tb[1 % 2].at[pl.ds(0, nrows)], msem)
        for t in range(1, NW):
            if t + 1 < NW:
                dnext = pltpu.async_copy(
                    tabs.at[pl.ds((t + 1) * m + r0, nrows)],
                    tb[(t + 1) % 2].at[pl.ds(0, nrows)], msem)
            dprev.wait()
            cur = tb[t % 2]

            def mx(j, c2, cur=cur):
                for k in range(unr):
                    o = (j * unr + k) * L
                    acc[pl.ds(o, L)] = jnp.maximum(acc[pl.ds(o, L)],
                                                   cur[pl.ds(o, L)])
                return c2

            lax.fori_loop(0, nv // unr, mx, 0)
            if t + 1 < NW:
                dprev = dnext

        def clamp_body(i, carry):
            for k in range(unr):
                o = (i * unr + k) * L
                wc_flat[pl.ds(o, L)] = jnp.maximum(acc[pl.ds(o, L)], 0)
            return carry

        lax.fori_loop(0, nv // unr, clamp_body, 0)
        wout = pltpu.async_copy(acc.at[pl.ds(0, nrows)],
                                win_out.at[pl.ds(r0, nrows)], msem)

        chunks = make_chunks(nrows)
        vds = [pltpu.async_copy(vi.at[wc_flat.at[pl.ds(off, n)]],
                                vsel.at[pl.ds(off, n)], vsem)
               for off, n in chunks]
        for vd in vds:
            vd.wait()

        nch = len(chunks)
        gds = [None] * nch
        wds = [None] * nch
        dsts = [None] * nch
        for i in range(nch + LA):
            if i < nch:
                off, n = chunks[i]
                b = i % NBUF
                if i >= NBUF:
                    wds[i - NBUF].wait()
                dsts[i] = rows.at[b, pl.ds(0, n), :]
                gds[i] = pltpu.async_copy(
                    cfg.at[vsel.at[pl.ds(off, n)]], dsts[i], gsem)
            j = i - LA
            if 0 <= j < nch:
                off, n = chunks[j]
                gds[j].wait()
                wds[j] = pltpu.async_copy(
                    dsts[j], g_out.at[pl.ds(r0 + off, n), :], wsem)
        for j in range(max(nch - NBUF, 0), nch):
            wds[j].wait()
        wout.wait()

    def body(tabs, vi, cfg, g_out, win_out, acc, tb0, tb1, wc_flat, vsel,
             rows, msem, gsem, wsem, vsem):
        cid = lax.axis_index("c")
        sid = lax.axis_index("s")
        wid = sid * 2 + cid
        r0 = wid * rpw
        run_range(tabs, vi, cfg, g_out, win_out, r0, rpw,
                  acc, [tb0, tb1], wc_flat, vsel, rows,
                  msem, gsem, wsem, vsem)

        @pl.when(wid == 0)
        def _():
            run_range(tabs, vi, cfg, g_out, win_out, NW * rpw, rem0,
                      acc, [tb0, tb1], wc_flat, vsel, rows,
                      msem, gsem, wsem, vsem)

    return pl.kernel(
        body,
        out_type=[
            jax.ShapeDtypeStruct((m, d), jnp.float32),
            jax.ShapeDtypeStruct((m,), jnp.int32),
        ],
        mesh=mesh,
        scratch_types=[
            pltpu.VMEM((3120,), jnp.int32),
            pltpu.VMEM((3120,), jnp.int32),
            pltpu.VMEM((3120,), jnp.int32),
            pltpu.VMEM((3120,), jnp.int32),
            pltpu.VMEM((3120,), jnp.int32),
            pltpu.VMEM((NBUF, CHUNK, d), jnp.float32),
            pltpu.SemaphoreType.DMA,
            pltpu.SemaphoreType.DMA,
            pltpu.SemaphoreType.DMA,
            pltpu.SemaphoreType.DMA,
        ],
    )


def _dense_body(prev_ref, g_ref, win_ref, wzp_ref, wzu_ref, wc_ref,
                bz_ref, bc_ref, gamma_ref, beta_ref, out_ref):
    prev = prev_ref[...]
    g = g_ref[...]
    a = jnp.dot(prev, wzp_ref[...], preferred_element_type=jnp.float32)
    b = jnp.dot(g, wzu_ref[...], preferred_element_type=jnp.float32)
    c = jnp.dot(g, wc_ref[...], preferred_element_type=jnp.float32)
    z = jax.nn.sigmoid(a + b + bz_ref[...])
    cand = jax.nn.relu(c + bc_ref[...])
    new = z * prev + (1.0 - z) * cand
    mask = win_ref[...].reshape(-1, 1) >= 0
    row = jnp.where(mask, new, prev)
    mean = jnp.mean(row, axis=-1, keepdims=True)
    var = jnp.mean(jnp.square(row - mean), axis=-1, keepdims=True)
    out_ref[...] = (row - mean) * lax.rsqrt(var + EPS) * gamma_ref[...] + beta_ref[...]


def _dense_update(prev, g, winner, Wz_p, Wz_u, bz, Wc, bc, gamma, beta):
    m, d = prev.shape
    grid = ((m + ROW_BLK - 1) // ROW_BLK,)
    row_spec = pl.BlockSpec((ROW_BLK, d), lambda i: (i, 0))
    win_spec = pl.BlockSpec((ROW_BLK,), lambda i: (i,))
    full_w = pl.BlockSpec((d, d), lambda i: (0, 0))
    vec_spec = pl.BlockSpec((1, d), lambda i: (0, 0))
    return pl.pallas_call(
        _dense_body,
        grid=grid,
        in_specs=[row_spec, row_spec, win_spec, full_w, full_w, full_w,
                  vec_spec, vec_spec, vec_spec, vec_spec],
        out_specs=row_spec,
        out_shape=jax.ShapeDtypeStruct((m, d), jnp.float32),
    )(prev, g, winner, Wz_p, Wz_u, Wc,
      bz[None, :], bc[None, :], gamma[None, :], beta[None, :])


def kernel(previous_ast_nodes_encodings, new_cfg_nodes_encodings,
           Wz_p, Wz_u, bz, Wc, bc, gamma, beta,
           key_indices, value_indices):
    prev = previous_ast_nodes_encodings
    cfg = new_cfg_nodes_encodings
    m, d = prev.shape
    ncfg = cfg.shape[0]
    u = key_indices.shape[0]
    ki = key_indices.astype(jnp.int32)
    vi = value_indices.astype(jnp.int32)

    tabs = _sc_dedup_build(m, u)(ki)
    g, win_merged = _sc_gather_build(m, d, u, ncfg)(tabs, vi, cfg)
    return _dense_update(prev, g, win_merged, Wz_p, Wz_u, bz, Wc, bc,
                         gamma, beta)


# speedup vs baseline: 1.2954x; 1.0380x over previous
import functools

import jax
import jax.numpy as jnp
from jax import lax
from jax.experimental import pallas as pl
from jax.experimental.pallas import tpu as pltpu
from jax.experimental.pallas import tpu_sc as plsc

EPS = 1e-5
ROW_BLK = 5120
L = 16
NW = 32
NW_ACT = 25
CHUNK = 128


def _sc_dedup_build(m, u):
    nvec = u // L
    vpw = nvec // NW
    extra = nvec - vpw * NW
    vchunk = 122
    nchunks = vpw // vchunk
    kpc = vchunk * L
    mgroups = m // L
    gpt = mgroups // 16
    rem_g = mgroups - gpt * 16
    wpt = gpt * L

    mesh = plsc.VectorSubcoreMesh(core_axis_name="c", subcore_axis_name="s")

    def process_vec(table, kbuf, voff, base_u):
        lanes = lax.broadcasted_iota(jnp.int32, (L,), 0)
        kv = kbuf[pl.ds(voff * L, L)]
        comp = (kv << 4) | lanes
        scomp, slane = plsc.sort_key_val(comp, lanes)
        skey = scomp >> 4
        uval = slane + base_u
        nxt = lax.gather(
            skey, jnp.minimum(lanes + 1, 15)[:, None],
            dimension_numbers=lax.GatherDimensionNumbers(
                offset_dims=(), collapsed_slice_dims=(0,),
                start_index_map=(0,)),
            slice_sizes=(1,), mode=lax.GatherScatterMode.PROMISE_IN_BOUNDS)
        end = (skey != nxt) | (lanes == 15)
        plsc.store_scatter(table, [skey], uval, mask=end)

    def body(ki, tabs_out, table, kbuf, sem):
        cid = lax.axis_index("c")
        sid = lax.axis_index("s")
        wid = sid * 2 + cid

        neg1 = jnp.full((L,), -1, jnp.int32)

        def memset(i, carry):
            for k in range(25):
                table[pl.ds((i * 25 + k) * L, L)] = neg1
            return carry

        lax.fori_loop(0, m // (L * 25), memset, 0)

        start_vec = wid * vpw + jnp.minimum(wid, extra)
        start_key = pl.multiple_of(start_vec * L, 8)
        for ch in range(nchunks):
            off = pl.multiple_of(start_key + ch * kpc, 8)
            pltpu.sync_copy(ki.at[pl.ds(off, kpc)], kbuf)

            def pv(j, carry):
                process_vec(table, kbuf, j, off + j * L)
                return carry

            lax.fori_loop(0, vchunk, pv, 0)

        @pl.when(wid < extra)
        def _():
            off = pl.multiple_of(start_key + vpw * L, 8)
            pltpu.sync_copy(ki.at[pl.ds(off, L)], kbuf.at[pl.ds(0, L)])
            process_vec(table, kbuf, 0, off)

        pltpu.sync_copy(table, tabs_out.at[pl.ds(wid * m, m)])

    return pl.kernel(
        body,
        out_type=jax.ShapeDtypeStruct((NW * m,), jnp.int32),
        mesh=mesh,
        compiler_params=pltpu.CompilerParams(needs_layout_passes=False),
        scratch_types=[
            pltpu.VMEM((m,), jnp.int32),
            pltpu.VMEM((kpc,), jnp.int32),
            pltpu.SemaphoreType.DMA,
        ],
    )


def _sc_gather_build(m, d, u, ncfg, r_base, nrows_total):
    rpw = (nrows_total // NW) // 8 * 8
    rem0 = nrows_total - NW * rpw
    mesh = plsc.VectorSubcoreMesh(core_axis_name="c", subcore_axis_name="s")

    NBUF = 4
    LA = 2

    def make_chunks(nrows):
        chunks = []
        off = 0
        while off < nrows:
            n = min(CHUNK, nrows - off)
            chunks.append((off, n))
            off += n
        return chunks

    def run_range(tabs, vi, cfg, g_out, win_out, rel0, nrows,
                  acc, tb, wc_flat, vsel, rows, msem, gsem, wsem, vsem):
        r0 = r_base + rel0
        pltpu.sync_copy(tabs.at[pl.ds(r0, nrows)], acc.at[pl.ds(0, nrows)])
        nv = nrows // L
        unr = 13 if nv % 13 == 0 else (10 if nv % 10 == 0 else 1)

        dprev = pltpu.async_copy(tabs.at[pl.ds(1 * m + r0, nrows)],
                                 tb[1 % 2].at[pl.ds(0, nrows)], msem)
        for t in range(1, NW):
            if t + 1 < NW:
                dnext = pltpu.async_copy(
                    tabs.at[pl.ds((t + 1) * m + r0, nrows)],
                    tb[(t + 1) % 2].at[pl.ds(0, nrows)], msem)
            dprev.wait()
            cur = tb[t % 2]

            def mx(j, c2, cur=cur):
                for k in range(unr):
                    o = (j * unr + k) * L
                    acc[pl.ds(o, L)] = jnp.maximum(acc[pl.ds(o, L)],
                                                   cur[pl.ds(o, L)])
                return c2

            lax.fori_loop(0, nv // unr, mx, 0)
            if t + 1 < NW:
                dprev = dnext

        def clamp_body(i, carry):
            for k in range(unr):
                o = (i * unr + k) * L
                wc_flat[pl.ds(o, L)] = jnp.maximum(acc[pl.ds(o, L)], 0)
            return carry

        lax.fori_loop(0, nv // unr, clamp_body, 0)
        wout = pltpu.async_copy(acc.at[pl.ds(0, nrows)],
                                win_out.at[pl.ds(rel0, nrows)], msem)

        chunks = make_chunks(nrows)
        vds = [pltpu.async_copy(vi.at[wc_flat.at[pl.ds(off, n)]],
                                vsel.at[pl.ds(off, n)], vsem)
               for off, n in chunks]
        for vd in vds:
            vd.wait()

        nch = len(chunks)
        gds = [None] * nch
        wds = [None] * nch
        dsts = [None] * nch
        for i in range(nch + LA):
            if i < nch:
                off, n = chunks[i]
                b = i % NBUF
                if i >= NBUF:
                    wds[i - NBUF].wait()
                dsts[i] = rows.at[b, pl.ds(0, n), :]
                gds[i] = pltpu.async_copy(
                    cfg.at[vsel.at[pl.ds(off, n)]], dsts[i], gsem)
            j = i - LA
            if 0 <= j < nch:
                off, n = chunks[j]
                gds[j].wait()
                wds[j] = pltpu.async_copy(
                    dsts[j], g_out.at[pl.ds(rel0 + off, n), :], wsem)
        for j in range(max(nch - NBUF, 0), nch):
            wds[j].wait()
        wout.wait()

    def body(tabs, vi, cfg, g_out, win_out, acc, tb0, tb1, wc_flat, vsel,
             rows, msem, gsem, wsem, vsem):
        cid = lax.axis_index("c")
        sid = lax.axis_index("s")
        wid = sid * 2 + cid
        run_range(tabs, vi, cfg, g_out, win_out, wid * rpw, rpw,
                  acc, [tb0, tb1], wc_flat, vsel, rows,
                  msem, gsem, wsem, vsem)

        if rem0:
            @pl.when(wid == 0)
            def _():
                run_range(tabs, vi, cfg, g_out, win_out, NW * rpw, rem0,
                          acc, [tb0, tb1], wc_flat, vsel, rows,
                          msem, gsem, wsem, vsem)

    return pl.kernel(
        body,
        out_type=[
            jax.ShapeDtypeStruct((nrows_total, d), jnp.float32),
            jax.ShapeDtypeStruct((nrows_total,), jnp.int32),
        ],
        mesh=mesh,
        scratch_types=[
            pltpu.VMEM((rpw,), jnp.int32),
            pltpu.VMEM((rpw,), jnp.int32),
            pltpu.VMEM((rpw,), jnp.int32),
            pltpu.VMEM((rpw,), jnp.int32),
            pltpu.VMEM((rpw,), jnp.int32),
            pltpu.VMEM((NBUF, CHUNK, d), jnp.float32),
            pltpu.SemaphoreType.DMA,
            pltpu.SemaphoreType.DMA,
            pltpu.SemaphoreType.DMA,
            pltpu.SemaphoreType.DMA,
        ],
    )


def _dense_body(prev_ref, g_ref, win_ref, wzp_ref, wzu_ref, wc_ref,
                bz_ref, bc_ref, gamma_ref, beta_ref, out_ref):
    prev = prev_ref[...]
    g = g_ref[...]
    a = jnp.dot(prev, wzp_ref[...], preferred_element_type=jnp.float32)
    b = jnp.dot(g, wzu_ref[...], preferred_element_type=jnp.float32)
    c = jnp.dot(g, wc_ref[...], preferred_element_type=jnp.float32)
    z = jax.nn.sigmoid(a + b + bz_ref[...])
    cand = jax.nn.relu(c + bc_ref[...])
    new = z * prev + (1.0 - z) * cand
    mask = win_ref[...].reshape(-1, 1) >= 0
    row = jnp.where(mask, new, prev)
    mean = jnp.mean(row, axis=-1, keepdims=True)
    var = jnp.mean(jnp.square(row - mean), axis=-1, keepdims=True)
    out_ref[...] = (row - mean) * lax.rsqrt(var + EPS) * gamma_ref[...] + beta_ref[...]


def _dense_update(prev, g, winner, Wz_p, Wz_u, bz, Wc, bc, gamma, beta,
                  blk_off=0, prior=None):
    m, d = prev.shape
    nrows = g.shape[0]
    grid = ((nrows + ROW_BLK - 1) // ROW_BLK,)
    off_spec = pl.BlockSpec((ROW_BLK, d), lambda i: (i + blk_off, 0))
    rel_spec = pl.BlockSpec((ROW_BLK, d), lambda i: (i, 0))
    win_spec = pl.BlockSpec((ROW_BLK,), lambda i: (i,))
    full_w = pl.BlockSpec((d, d), lambda i: (0, 0))
    vec_spec = pl.BlockSpec((1, d), lambda i: (0, 0))
    tiny = pl.BlockSpec((8, d), lambda i: (0, 0))
    if prior is None:
        in_specs = [off_spec, rel_spec, win_spec, full_w, full_w, full_w,
                    vec_spec, vec_spec, vec_spec, vec_spec]
        args = (prev, g, winner, Wz_p, Wz_u, Wc,
                bz[None, :], bc[None, :], gamma[None, :], beta[None, :])
        aliases = {}
        body = _dense_body
    else:
        in_specs = [tiny, off_spec, rel_spec, win_spec, full_w, full_w,
                    full_w, vec_spec, vec_spec, vec_spec, vec_spec]
        args = (prior, prev, g, winner, Wz_p, Wz_u, Wc,
                bz[None, :], bc[None, :], gamma[None, :], beta[None, :])
        aliases = {0: 0}

        def body(prior_ref, *rest):
            _dense_body(*rest)

    return pl.pallas_call(
        body,
        grid=grid,
        in_specs=in_specs,
        out_specs=off_spec,
        out_shape=jax.ShapeDtypeStruct((m, d), jnp.float32),
        input_output_aliases=aliases,
    )(*args)


def kernel(previous_ast_nodes_encodings, new_cfg_nodes_encodings,
           Wz_p, Wz_u, bz, Wc, bc, gamma, beta,
           key_indices, value_indices):
    prev = previous_ast_nodes_encodings
    cfg = new_cfg_nodes_encodings
    m, d = prev.shape
    ncfg = cfg.shape[0]
    u = key_indices.shape[0]
    ki = key_indices.astype(jnp.int32)
    vi = value_indices.astype(jnp.int32)

    tabs = _sc_dedup_build(m, u)(ki)
    split = 10 * ROW_BLK
    g_a, win_a = _sc_gather_build(m, d, u, ncfg, 0, split)(tabs, vi, cfg)
    g_b, win_b = _sc_gather_build(m, d, u, ncfg, split, m - split)(
        tabs, vi, cfg)
    out_a = _dense_update(prev, g_a, win_a, Wz_p, Wz_u, bz, Wc, bc,
                          gamma, beta, blk_off=0)
    return _dense_update(prev, g_b, win_b, Wz_p, Wz_u, bz, Wc, bc,
                         gamma, beta, blk_off=split // ROW_BLK, prior=out_a)
